# Initial kernel scaffold; baseline (speedup 1.0000x reference)
import jax
import jax.numpy as jnp
from jax.experimental import pallas as pl


def kernel(z, pos_edge_index, neg_edge_index):
    raise NotImplementedError("write your pallas kernel here")


# speedup vs baseline: 2.5295x; 2.5295x over previous
import functools

import jax
import jax.numpy as jnp
from jax import lax
from jax.experimental import pallas as pl
from jax.experimental.pallas import tpu as pltpu
from jax.experimental.pallas import tpu_sc as plsc

EPS = 1e-15
N_NODES = 10000
D_FEAT = 128
N_EDGES = 320000

NUM_CORES = 2
NUM_SUBCORES = 16
NUM_WORKERS = NUM_CORES * NUM_SUBCORES
EDGES_PER_WORKER = N_EDGES // NUM_WORKERS
CHUNK = 80
NUM_CHUNKS = EDGES_PER_WORKER // CHUNK
LANES = 16
D_CHUNKS = D_FEAT // LANES


def _sc_body(z_hbm, ps_hbm, pd_hbm, ns_hbm, nd_hbm, pos_out, neg_out,
             sidx, didx, srows, drows, outv, sem):
    wid = lax.axis_index("s") * NUM_CORES + lax.axis_index("c")

    def process(src_hbm, dst_hbm, out_hbm):
        def chunk_body(c, carry):
            base = wid * EDGES_PER_WORKER + c * CHUNK
            pltpu.sync_copy(src_hbm.at[pl.ds(base, CHUNK)], sidx)
            pltpu.sync_copy(dst_hbm.at[pl.ds(base, CHUNK)], didx)
            cp1 = pltpu.async_copy(z_hbm.at[sidx], srows, sem)
            cp2 = pltpu.async_copy(z_hbm.at[didx], drows, sem)
            cp1.wait()
            cp2.wait()

            def edge_body(e, carry2):
                acc = srows[e, pl.ds(0, LANES)] * drows[e, pl.ds(0, LANES)]
                for j in range(1, D_CHUNKS):
                    acc = acc + (srows[e, pl.ds(j * LANES, LANES)] *
                                 drows[e, pl.ds(j * LANES, LANES)])
                outv[e] = acc
                return carry2

            lax.fori_loop(0, CHUNK, edge_body, 0, unroll=4)
            pltpu.sync_copy(outv, out_hbm.at[pl.ds(base, CHUNK)])
            return carry

        lax.fori_loop(0, NUM_CHUNKS, chunk_body, 0)

    process(ps_hbm, pd_hbm, pos_out)
    process(ns_hbm, nd_hbm, neg_out)


_sc_dots = pl.kernel(
    _sc_body,
    out_type=(
        jax.ShapeDtypeStruct((N_EDGES, LANES), jnp.float32),
        jax.ShapeDtypeStruct((N_EDGES, LANES), jnp.float32),
    ),
    mesh=plsc.VectorSubcoreMesh(
        core_axis_name="c", subcore_axis_name="s",
        num_cores=NUM_CORES, num_subcores=NUM_SUBCORES,
    ),
    scratch_types=[
        pltpu.VMEM((CHUNK,), jnp.int32),
        pltpu.VMEM((CHUNK,), jnp.int32),
        pltpu.VMEM((CHUNK, D_FEAT), jnp.float32),
        pltpu.VMEM((CHUNK, D_FEAT), jnp.float32),
        pltpu.VMEM((CHUNK, LANES), jnp.float32),
        pltpu.SemaphoreType.DMA,
    ],
)


_R_TOTAL = N_EDGES * LANES // D_FEAT
_TC_STEPS = 8
_R_BLOCK = _R_TOTAL // _TC_STEPS


def _tc_loss_body(pos_ref, neg_ref, out_ref):
    i = pl.program_id(0)
    r = lax.broadcasted_iota(jnp.int32, (D_FEAT, D_FEAT // LANES), 0)
    c = lax.broadcasted_iota(jnp.int32, (D_FEAT, D_FEAT // LANES), 1)
    sel = jnp.where(r // LANES == c, 1.0, 0.0).astype(jnp.float32)
    p = jnp.dot(pos_ref[...], sel, preferred_element_type=jnp.float32)
    n = jnp.dot(neg_ref[...], sel, preferred_element_type=jnp.float32)
    pos_l = -jnp.log(jax.nn.sigmoid(p) + EPS)
    neg_l = -jnp.log(1.0 - jax.nn.sigmoid(n) + EPS)
    part = (jnp.sum(pos_l) + jnp.sum(neg_l)) / N_EDGES

    @pl.when(i == 0)
    def _init():
        out_ref[0, 0] = part

    @pl.when(i > 0)
    def _acc():
        out_ref[0, 0] += part


_tc_loss = pl.pallas_call(
    _tc_loss_body,
    grid=(_TC_STEPS,),
    in_specs=[
        pl.BlockSpec((_R_BLOCK, D_FEAT), lambda i: (i, 0)),
        pl.BlockSpec((_R_BLOCK, D_FEAT), lambda i: (i, 0)),
    ],
    out_specs=pl.BlockSpec(memory_space=pltpu.SMEM),
    out_shape=jax.ShapeDtypeStruct((1, 1), jnp.float32),
)


def kernel(z, pos_edge_index, neg_edge_index):
    ps = pos_edge_index[0].astype(jnp.int32)
    pd = pos_edge_index[1].astype(jnp.int32)
    ns = neg_edge_index[0].astype(jnp.int32)
    nd = neg_edge_index[1].astype(jnp.int32)
    pos_vals, neg_vals = _sc_dots(z, ps, pd, ns, nd)
    loss = _tc_loss(pos_vals.reshape(_R_TOTAL, D_FEAT),
                    neg_vals.reshape(_R_TOTAL, D_FEAT))
    return loss[0, 0]


# speedup vs baseline: 7.5696x; 2.9925x over previous
import functools

import jax
import jax.numpy as jnp
from jax import lax
from jax.experimental import pallas as pl
from jax.experimental.pallas import tpu as pltpu
from jax.experimental.pallas import tpu_sc as plsc

EPS = 1e-15
N_NODES = 10000
D_FEAT = 128
N_EDGES = 320000
E_TOTAL = 2 * N_EDGES

NUM_CORES = 2
NUM_SUBCORES = 16
NUM_WORKERS = NUM_CORES * NUM_SUBCORES
EDGES_PER_WORKER = E_TOTAL // NUM_WORKERS
CHUNK = 80
NUM_CHUNKS = EDGES_PER_WORKER // CHUNK
LANES = 16
D_CHUNKS = D_FEAT // LANES
EDGES_PER_ROW = D_FEAT // LANES
ROWS_PER_CHUNK = CHUNK // EDGES_PER_ROW
ROWS_PER_WORKER = EDGES_PER_WORKER // EDGES_PER_ROW
R_TOTAL = E_TOTAL // EDGES_PER_ROW


def _sc_body(z_hbm, src_hbm, dst_hbm, out_hbm,
             sidx, didx, srows0, srows1, drows0, drows1, outv0, outv1,
             gsem0, gsem1, wsem0, wsem1):
    wid = lax.axis_index("s") * NUM_CORES + lax.axis_index("c")
    ebase = wid * EDGES_PER_WORKER
    rbase = wid * ROWS_PER_WORKER

    pltpu.sync_copy(src_hbm.at[pl.ds(ebase, EDGES_PER_WORKER)], sidx)
    pltpu.sync_copy(dst_hbm.at[pl.ds(ebase, EDGES_PER_WORKER)], didx)

    bufs = ((srows0, drows0, outv0, gsem0, wsem0),
            (srows1, drows1, outv1, gsem1, wsem1))

    def gathers(c, b):
        srows, drows, _, gsem, _ = bufs[b]
        cs = pltpu.make_async_copy(z_hbm.at[sidx.at[pl.ds(c * CHUNK, CHUNK)]],
                                   srows, gsem)
        cd = pltpu.make_async_copy(z_hbm.at[didx.at[pl.ds(c * CHUNK, CHUNK)]],
                                   drows, gsem)
        return cs, cd

    def writeback(c, b):
        _, _, outv, _, wsem = bufs[b]
        return pltpu.make_async_copy(
            outv, out_hbm.at[pl.ds((ebase + c * CHUNK) * LANES,
                                   CHUNK * LANES)], wsem)

    def compute(b):
        srows, drows, outv, _, _ = bufs[b]

        def row_body(r, carry):
            for e8 in range(EDGES_PER_ROW):
                e = r * EDGES_PER_ROW + e8
                acc = (srows[e, pl.ds(0, LANES)] *
                       drows[e, pl.ds(0, LANES)])
                for j in range(1, D_CHUNKS):
                    acc = acc + (srows[e, pl.ds(j * LANES, LANES)] *
                                 drows[e, pl.ds(j * LANES, LANES)])
                outv[pl.ds(e * LANES, LANES)] = acc
            return carry

        lax.fori_loop(0, ROWS_PER_CHUNK, row_body, 0)

    def step(c, b, first):
        cs, cd = gathers(c, b)
        cs.wait()
        cd.wait()
        if not first:
            writeback(c - 2, b).wait()
        compute(b)
        writeback(c, b).start()
        return c

    for b in range(2):
        cs, cd = gathers(b, b)
        cs.start()
        cd.start()

    def pair(s, carry):
        c = 2 * s
        for b in range(2):
            step(c + b, b, first=False)
            cs, cd = gathers(c + b + 2, b)
            cs.start()
            cd.start()
        return carry

    for b in range(2):
        step(b, b, first=True)
        cs, cd = gathers(b + 2, b)
        cs.start()
        cd.start()
    lax.fori_loop(1, NUM_CHUNKS // 2 - 1, pair, 0)
    for b in range(2):
        step(NUM_CHUNKS - 2 + b, b, first=False)
    for b in range(2):
        writeback(NUM_CHUNKS - 2 + b, b).wait()


_sc_dots = pl.kernel(
    _sc_body,
    out_type=jax.ShapeDtypeStruct((E_TOTAL * LANES,), jnp.float32),
    mesh=plsc.VectorSubcoreMesh(
        core_axis_name="c", subcore_axis_name="s",
        num_cores=NUM_CORES, num_subcores=NUM_SUBCORES,
    ),
    scratch_types=[
        pltpu.VMEM((EDGES_PER_WORKER,), jnp.int32),
        pltpu.VMEM((EDGES_PER_WORKER,), jnp.int32),
        pltpu.VMEM((CHUNK, D_FEAT), jnp.float32),
        pltpu.VMEM((CHUNK, D_FEAT), jnp.float32),
        pltpu.VMEM((CHUNK, D_FEAT), jnp.float32),
        pltpu.VMEM((CHUNK, D_FEAT), jnp.float32),
        pltpu.VMEM((CHUNK * LANES,), jnp.float32),
        pltpu.VMEM((CHUNK * LANES,), jnp.float32),
        pltpu.SemaphoreType.DMA,
        pltpu.SemaphoreType.DMA,
        pltpu.SemaphoreType.DMA,
        pltpu.SemaphoreType.DMA,
    ],
)


_TC_STEPS = 8
_R_BLOCK = (R_TOTAL // 2) // _TC_STEPS


def _tc_loss_body(pos_ref, neg_ref, out_ref):
    i = pl.program_id(0)
    r = lax.broadcasted_iota(jnp.int32, (D_FEAT, EDGES_PER_ROW), 0)
    c = lax.broadcasted_iota(jnp.int32, (D_FEAT, EDGES_PER_ROW), 1)
    sel = jnp.where(r // LANES == c, 1.0, 0.0).astype(jnp.float32)
    p = jnp.dot(pos_ref[...], sel, preferred_element_type=jnp.float32)
    n = jnp.dot(neg_ref[...], sel, preferred_element_type=jnp.float32)
    pos_l = -jnp.log(jax.nn.sigmoid(p) + EPS)
    neg_l = -jnp.log(1.0 - jax.nn.sigmoid(n) + EPS)
    part = (jnp.sum(pos_l) + jnp.sum(neg_l)) / N_EDGES

    @pl.when(i == 0)
    def _init():
        out_ref[0, 0] = part

    @pl.when(i > 0)
    def _acc():
        out_ref[0, 0] += part


_tc_loss = pl.pallas_call(
    _tc_loss_body,
    grid=(_TC_STEPS,),
    in_specs=[
        pl.BlockSpec((_R_BLOCK, D_FEAT), lambda i: (i, 0)),
        pl.BlockSpec((_R_BLOCK, D_FEAT), lambda i: (i + _TC_STEPS, 0)),
    ],
    out_specs=pl.BlockSpec(memory_space=pltpu.SMEM),
    out_shape=jax.ShapeDtypeStruct((1, 1), jnp.float32),
)


def kernel(z, pos_edge_index, neg_edge_index):
    src = jnp.concatenate([pos_edge_index[0], neg_edge_index[0]]
                          ).astype(jnp.int32)
    dst = jnp.concatenate([pos_edge_index[1], neg_edge_index[1]]
                          ).astype(jnp.int32)
    vals = _sc_dots(z, src, dst).reshape(R_TOTAL, D_FEAT)
    loss = _tc_loss(vals, vals)
    return loss[0, 0]
